# baseline (speedup 1.0000x reference)
import functools

import jax
import jax.numpy as jnp
from jax import lax
from jax.experimental import pallas as pl
from jax.experimental.pallas import tpu as pltpu
from jax.experimental.pallas import tpu_sc as plsc

E = 16
BLK = 128
NT = 32
LANES = 16


def _cumsum_rows(m, carry):
    n = m.shape[0]
    ch = 512
    r = lax.broadcasted_iota(jnp.int32, (ch, ch), 0)
    c = lax.broadcasted_iota(jnp.int32, (ch, ch), 1)
    lt = (r >= c).astype(jnp.float32)
    chunks = []
    for i in range(n // ch):
        blk = m[i * ch:(i + 1) * ch, :]
        cs = jnp.dot(lt, blk, preferred_element_type=jnp.float32) + carry
        carry = cs[ch - 1:ch, :]
        chunks.append(cs)
    return jnp.concatenate(chunks, axis=0), carry


def _router_body(x_ref, gate_ref, swg_ref, swu_ref, swd_ref,
                 pos_ref, wts_ref, be_ref, shared_ref):
    x = x_ref[...]
    T = x.shape[0]
    logits = jnp.dot(x, gate_ref[...], preferred_element_type=jnp.float32)
    ecols = lax.broadcasted_iota(jnp.int32, logits.shape, 1)
    l1 = jnp.max(logits, axis=-1, keepdims=True)
    i1 = jnp.min(jnp.where(logits == l1, ecols, E), axis=-1, keepdims=True)
    masked = jnp.where(ecols == i1, -jnp.inf, logits)
    l2 = jnp.max(masked, axis=-1, keepdims=True)
    i2 = jnp.min(jnp.where(masked == l2, ecols, E), axis=-1, keepdims=True)
    w1 = 1.0 / (1.0 + jnp.exp(l2 - l1))
    w2 = 1.0 - w1

    m0 = (ecols == i1).astype(jnp.float32)
    m1 = (ecols == i2).astype(jnp.float32)
    zero = jnp.zeros((1, E), jnp.float32)
    c0, carry0 = _cumsum_rows(m0, zero)
    c1, hist = _cumsum_rows(m1, carry0)
    padded = (((hist.astype(jnp.int32) + (BLK - 1)) // BLK) * BLK)
    padded = padded.astype(jnp.float32)
    er = lax.broadcasted_iota(jnp.int32, (E, E), 0)
    ec = lax.broadcasted_iota(jnp.int32, (E, E), 1)
    ut = (er < ec).astype(jnp.float32)
    offs = jnp.dot(padded, ut, preferred_element_type=jnp.float32)

    rank0 = jnp.sum(c0 * m0, axis=1) - 1.0
    rank1 = jnp.sum(c1 * m1, axis=1) - 1.0
    off0 = jnp.sum(offs * m0, axis=1)
    off1 = jnp.sum(offs * m1, axis=1)
    pos_ref[0, :] = (rank0 + off0).astype(jnp.int32)
    pos_ref[1, :] = (rank1 + off1).astype(jnp.int32)
    wts_ref[0, :] = w1[:, 0]
    wts_ref[1, :] = w2[:, 0]

    nb = be_ref.shape[0]
    bstart = (lax.broadcasted_iota(jnp.int32, (nb, E), 0) * BLK
              ).astype(jnp.float32)
    eids = lax.broadcasted_iota(jnp.int32, (nb, E), 1)
    inb = (bstart >= offs) & (bstart < offs + padded)
    be_ref[...] = jnp.sum(jnp.where(inb, eids, 0), axis=1).astype(jnp.int32)

    xb = x.astype(jnp.bfloat16)
    sg = jnp.dot(xb, swg_ref[...].astype(jnp.bfloat16),
                 preferred_element_type=jnp.float32)
    su = jnp.dot(xb, swu_ref[...].astype(jnp.bfloat16),
                 preferred_element_type=jnp.float32)
    sh = ((sg * jax.nn.sigmoid(sg)) * su).astype(jnp.bfloat16)
    shared_ref[...] = jnp.dot(sh, swd_ref[...].astype(jnp.bfloat16),
                              preferred_element_type=jnp.float32)


def _dispatch_body(x_hbm, pos_hbm, xs_hbm, idx_v, rows_v, sem):
    tpc = x_hbm.shape[0] // (NT // 2)
    wid = lax.axis_index("s") * 2 + lax.axis_index("c")
    tokbase = (wid % (NT // 2)) * tpc
    pltpu.sync_copy(pos_hbm.at[pl.ds(wid * tpc, tpc)], idx_v)
    pltpu.sync_copy(x_hbm.at[pl.ds(tokbase, tpc)], rows_v)
    pltpu.async_copy(rows_v, xs_hbm.at[idx_v], sem).wait()


def _expert_body(be_ref, xs_ref, wg_ref, wu_ref, wd_ref, ys_ref):
    xb = xs_ref[...].astype(jnp.bfloat16)
    g = jnp.dot(xb, wg_ref[0].astype(jnp.bfloat16),
                preferred_element_type=jnp.float32)
    u = jnp.dot(xb, wu_ref[0].astype(jnp.bfloat16),
                preferred_element_type=jnp.float32)
    h = ((g * jax.nn.sigmoid(g)) * u).astype(jnp.bfloat16)
    ys_ref[...] = jnp.dot(h, wd_ref[0].astype(jnp.bfloat16),
                          preferred_element_type=jnp.float32)


def _combine_body(shared_hbm, ys_hbm, pos_hbm, wts_hbm, out_hbm,
                  bufa_v, bufb_v, idx_v, w_v, sem):
    T = shared_hbm.shape[0]
    D = shared_hbm.shape[1]
    tpc = T // NT
    wid = lax.axis_index("s") * 2 + lax.axis_index("c")
    tokbase = wid * tpc
    pltpu.sync_copy(shared_hbm.at[pl.ds(tokbase, tpc)], bufa_v)

    for k in range(2):
        pltpu.sync_copy(pos_hbm.at[pl.ds(k * T + tokbase, tpc)], idx_v)
        pltpu.sync_copy(wts_hbm.at[pl.ds(k * T + tokbase, tpc)], w_v)
        pltpu.async_copy(ys_hbm.at[idx_v], bufb_v, sem).wait()
        def _grp(g, _):
            w16 = w_v[pl.ds(g * LANES, LANES)]
            for ii in range(LANES):
                wi = jnp.broadcast_to(w16[ii], (LANES,))
                row = g * LANES + ii
                for j in range(D // LANES):
                    sl = pl.ds(j * LANES, LANES)
                    bufa_v[row, sl] = bufa_v[row, sl] + wi * bufb_v[row, sl]
            return 0
        lax.fori_loop(0, tpc // LANES, _grp, 0)

    pltpu.sync_copy(bufa_v, out_hbm.at[pl.ds(tokbase, tpc)])


def _run_router(x, gate_w, shared_wg, shared_wu, shared_wd):
    T, D = x.shape
    SF = shared_wg.shape[-1]
    S = 2 * T
    P = S + E * BLK
    NB = P // BLK

    return pl.pallas_call(
        _router_body,
        grid=(1,),
        in_specs=[
            pl.BlockSpec((T, D), lambda i: (0, 0)),
            pl.BlockSpec((D, E), lambda i: (0, 0)),
            pl.BlockSpec((D, SF), lambda i: (0, 0)),
            pl.BlockSpec((D, SF), lambda i: (0, 0)),
            pl.BlockSpec((SF, D), lambda i: (0, 0)),
        ],
        out_specs=[
            pl.BlockSpec((2, T), lambda i: (0, 0)),
            pl.BlockSpec((2, T), lambda i: (0, 0)),
            pl.BlockSpec((NB,), lambda i: (0,)),
            pl.BlockSpec((T, D), lambda i: (0, 0)),
        ],
        out_shape=[
            jax.ShapeDtypeStruct((2, T), jnp.int32),
            jax.ShapeDtypeStruct((2, T), jnp.float32),
            jax.ShapeDtypeStruct((NB,), jnp.int32),
            jax.ShapeDtypeStruct((T, D), jnp.float32),
        ],
    )(x, gate_w, shared_wg, shared_wu, shared_wd)


def _run_dispatch(x, pos_flat):
    T, D = x.shape
    S = 2 * T
    P = S + E * BLK
    mesh = plsc.VectorSubcoreMesh(core_axis_name="c", subcore_axis_name="s",
                                  num_cores=2, num_subcores=NT // 2)
    tpc = T // (NT // 2)
    return pl.kernel(
        _dispatch_body,
        out_type=jax.ShapeDtypeStruct((P, D), jnp.float32),
        mesh=mesh,
        scratch_types=[
            pltpu.VMEM((tpc,), jnp.int32),
            pltpu.VMEM((tpc, D), jnp.float32),
            pltpu.SemaphoreType.DMA,
        ],
    )(x, pos_flat)


def _run_experts(be, xs, expert_wg, expert_wu, expert_wd):
    P, D = xs.shape
    F = expert_wg.shape[-1]
    NB = P // BLK
    return pl.pallas_call(
        _expert_body,
        grid_spec=pltpu.PrefetchScalarGridSpec(
            num_scalar_prefetch=1,
            grid=(NB,),
            in_specs=[
                pl.BlockSpec((BLK, D), lambda b, be_s: (b, 0)),
                pl.BlockSpec((1, D, F), lambda b, be_s: (be_s[b], 0, 0)),
                pl.BlockSpec((1, D, F), lambda b, be_s: (be_s[b], 0, 0)),
                pl.BlockSpec((1, F, D), lambda b, be_s: (be_s[b], 0, 0)),
            ],
            out_specs=pl.BlockSpec((BLK, D), lambda b, be_s: (b, 0)),
        ),
        out_shape=jax.ShapeDtypeStruct((P, D), jnp.float32),
        compiler_params=pltpu.CompilerParams(
            dimension_semantics=("arbitrary",),
        ),
    )(be, xs, expert_wg, expert_wu, expert_wd)


def _run_combine(shared, ys, pos_flat, wts_flat):
    T, D = shared.shape
    mesh = plsc.VectorSubcoreMesh(core_axis_name="c", subcore_axis_name="s",
                                  num_cores=2, num_subcores=NT // 2)
    ctpc = T // NT
    return pl.kernel(
        _combine_body,
        out_type=jax.ShapeDtypeStruct((T, D), jnp.float32),
        mesh=mesh,
        scratch_types=[
            pltpu.VMEM((ctpc, D), jnp.float32),
            pltpu.VMEM((ctpc, D), jnp.float32),
            pltpu.VMEM((ctpc,), jnp.int32),
            pltpu.VMEM((ctpc,), jnp.float32),
            pltpu.SemaphoreType.DMA,
        ],
    )(shared, ys, pos_flat, wts_flat)


def kernel(hidden_states, gate_w, expert_wg, expert_wu, expert_wd,
           shared_wg, shared_wu, shared_wd):
    orig_shape = hidden_states.shape
    D = orig_shape[-1]
    x = hidden_states.reshape(-1, D)
    T = x.shape[0]
    S = 2 * T
    P = S + E * BLK

    pos, wts, be, shared = _run_router(x, gate_w, shared_wg, shared_wu,
                                       shared_wd)
    xs = _run_dispatch(x, pos.reshape(S))
    ys = _run_experts(be, xs, expert_wg, expert_wu, expert_wd)
    out = _run_combine(shared, ys, pos.reshape(S), wts.reshape(S))
    return out.reshape(orig_shape)


# speedup vs baseline: 2.3381x; 2.3381x over previous
import jax
import jax.numpy as jnp
from jax.experimental import pallas as pl
from jax.experimental.pallas import tpu as pltpu

E = 16
TOPK = 2


def _moe_body(x_ref, gate_ref, wg_ref, wu_ref, wd_ref, swg_ref, swu_ref, swd_ref,
              out_ref, comb_ref, xb_ref):
    e = pl.program_id(0)

    @pl.when(e == 0)
    def _init():
        x = x_ref[...]
        xb_ref[...] = x.astype(jnp.bfloat16)
        logits = jnp.dot(x, gate_ref[...], preferred_element_type=jnp.float32)
        ecols = jax.lax.broadcasted_iota(jnp.int32, logits.shape, 1)
        l1 = jnp.max(logits, axis=-1, keepdims=True)
        i1 = jnp.min(jnp.where(logits == l1, ecols, E), axis=-1, keepdims=True)
        masked = jnp.where(ecols == i1, -jnp.inf, logits)
        l2 = jnp.max(masked, axis=-1, keepdims=True)
        i2 = jnp.min(jnp.where(masked == l2, ecols, E), axis=-1, keepdims=True)
        w1 = 1.0 / (1.0 + jnp.exp(l2 - l1))
        w2 = 1.0 - w1
        comb_ref[...] = jnp.where(ecols == i1, w1, 0.0) + jnp.where(ecols == i2, w2, 0.0)

        xb0 = xb_ref[...]
        sg = jnp.dot(xb0, swg_ref[...].astype(jnp.bfloat16),
                     preferred_element_type=jnp.float32)
        su = jnp.dot(xb0, swu_ref[...].astype(jnp.bfloat16),
                     preferred_element_type=jnp.float32)
        sh = ((sg * jax.nn.sigmoid(sg)) * su).astype(jnp.bfloat16)
        out_ref[...] = jnp.dot(sh, swd_ref[...].astype(jnp.bfloat16),
                               preferred_element_type=jnp.float32)

    xb = xb_ref[...]
    ecols = jax.lax.broadcasted_iota(jnp.int32, comb_ref.shape, 1)
    acc = out_ref[...]
    for sub in range(2):
        ee = 2 * e + sub
        g = jnp.dot(xb, wg_ref[sub].astype(jnp.bfloat16),
                    preferred_element_type=jnp.float32)
        u = jnp.dot(xb, wu_ref[sub].astype(jnp.bfloat16),
                    preferred_element_type=jnp.float32)
        coef = jnp.sum(jnp.where(ecols == ee, comb_ref[...], 0.0),
                       axis=-1, keepdims=True)
        h = (coef * (g * jax.nn.sigmoid(g)) * u).astype(jnp.bfloat16)
        acc += jnp.dot(h, wd_ref[sub].astype(jnp.bfloat16),
                       preferred_element_type=jnp.float32)
    out_ref[...] = acc


def kernel(hidden_states, gate_w, expert_wg, expert_wu, expert_wd,
           shared_wg, shared_wu, shared_wd):
    orig_shape = hidden_states.shape
    D = orig_shape[-1]
    x = hidden_states.reshape(-1, D)
    T = x.shape[0]
    F = expert_wg.shape[-1]
    SF = shared_wg.shape[-1]

    out = pl.pallas_call(
        _moe_body,
        grid=(E // 2,),
        in_specs=[
            pl.BlockSpec((T, D), lambda e: (0, 0)),
            pl.BlockSpec((D, E), lambda e: (0, 0)),
            pl.BlockSpec((2, D, F), lambda e: (e, 0, 0)),
            pl.BlockSpec((2, D, F), lambda e: (e, 0, 0)),
            pl.BlockSpec((2, F, D), lambda e: (e, 0, 0)),
            pl.BlockSpec((D, SF), lambda e: (0, 0)),
            pl.BlockSpec((D, SF), lambda e: (0, 0)),
            pl.BlockSpec((SF, D), lambda e: (0, 0)),
        ],
        out_specs=pl.BlockSpec((T, D), lambda e: (0, 0)),
        out_shape=jax.ShapeDtypeStruct((T, D), jnp.float32),
        scratch_shapes=[pltpu.VMEM((T, E), jnp.float32),
                        pltpu.VMEM((T, D), jnp.bfloat16)],
        compiler_params=pltpu.CompilerParams(
            dimension_semantics=("arbitrary",),
        ),
    )(x, gate_w, expert_wg, expert_wu, expert_wd, shared_wg, shared_wu, shared_wd)

    return out.reshape(orig_shape)


# speedup vs baseline: 2.4089x; 1.0303x over previous
import jax
import jax.numpy as jnp
from jax.experimental import pallas as pl
from jax.experimental.pallas import tpu as pltpu

E = 16
TOPK = 2


def _moe_body(x_ref, gate_ref, wg_ref, wu_ref, wd_ref, swg_ref, swu_ref, swd_ref,
              out_ref, comb_ref, xb_ref, acc_ref):
    e = pl.program_id(0)

    @pl.when(e == 0)
    def _init():
        x = x_ref[...]
        xb_ref[...] = x.astype(jnp.bfloat16)
        acc_ref[...] = jnp.zeros_like(acc_ref)
        logits = jnp.dot(x, gate_ref[...], preferred_element_type=jnp.float32)
        ecols = jax.lax.broadcasted_iota(jnp.int32, logits.shape, 1)
        l1 = jnp.max(logits, axis=-1, keepdims=True)
        i1 = jnp.min(jnp.where(logits == l1, ecols, E), axis=-1, keepdims=True)
        masked = jnp.where(ecols == i1, -jnp.inf, logits)
        l2 = jnp.max(masked, axis=-1, keepdims=True)
        i2 = jnp.min(jnp.where(masked == l2, ecols, E), axis=-1, keepdims=True)
        w1 = 1.0 / (1.0 + jnp.exp(l2 - l1))
        w2 = 1.0 - w1
        comb_ref[...] = jnp.where(ecols == i1, w1, 0.0) + jnp.where(ecols == i2, w2, 0.0)

        xb0 = xb_ref[...]
        sg = jnp.dot(xb0, swg_ref[...].astype(jnp.bfloat16),
                     preferred_element_type=jnp.float32)
        su = jnp.dot(xb0, swu_ref[...].astype(jnp.bfloat16),
                     preferred_element_type=jnp.float32)
        sh = ((sg * jax.nn.sigmoid(sg)) * su).astype(jnp.bfloat16)
        out_ref[...] = jnp.dot(sh, swd_ref[...].astype(jnp.bfloat16),
                               preferred_element_type=jnp.float32)

    xb = xb_ref[...]
    g = jnp.dot(xb, wg_ref[0].astype(jnp.bfloat16), preferred_element_type=jnp.float32)
    u = jnp.dot(xb, wu_ref[0].astype(jnp.bfloat16), preferred_element_type=jnp.float32)
    ecols = jax.lax.broadcasted_iota(jnp.int32, comb_ref.shape, 1)
    coef = jnp.sum(jnp.where(ecols == e, comb_ref[...], 0.0), axis=-1, keepdims=True)
    h = (coef * (g * jax.nn.sigmoid(g)) * u).astype(jnp.bfloat16)
    acc_ref[...] += jnp.dot(h, wd_ref[0].astype(jnp.bfloat16),
                            preferred_element_type=jnp.float32
                            ).astype(jnp.bfloat16)

    @pl.when(e == E - 1)
    def _final():
        out_ref[...] += acc_ref[...].astype(jnp.float32)


def kernel(hidden_states, gate_w, expert_wg, expert_wu, expert_wd,
           shared_wg, shared_wu, shared_wd):
    orig_shape = hidden_states.shape
    D = orig_shape[-1]
    x = hidden_states.reshape(-1, D)
    T = x.shape[0]
    F = expert_wg.shape[-1]
    SF = shared_wg.shape[-1]

    out = pl.pallas_call(
        _moe_body,
        grid=(E,),
        in_specs=[
            pl.BlockSpec((T, D), lambda e: (0, 0)),
            pl.BlockSpec((D, E), lambda e: (0, 0)),
            pl.BlockSpec((1, D, F), lambda e: (e, 0, 0)),
            pl.BlockSpec((1, D, F), lambda e: (e, 0, 0)),
            pl.BlockSpec((1, F, D), lambda e: (e, 0, 0)),
            pl.BlockSpec((D, SF), lambda e: (0, 0)),
            pl.BlockSpec((D, SF), lambda e: (0, 0)),
            pl.BlockSpec((SF, D), lambda e: (0, 0)),
        ],
        out_specs=pl.BlockSpec((T, D), lambda e: (0, 0)),
        out_shape=jax.ShapeDtypeStruct((T, D), jnp.float32),
        scratch_shapes=[pltpu.VMEM((T, E), jnp.float32),
                        pltpu.VMEM((T, D), jnp.bfloat16),
                        pltpu.VMEM((T, D), jnp.bfloat16)],
        compiler_params=pltpu.CompilerParams(
            dimension_semantics=("arbitrary",),
        ),
    )(x, gate_w, expert_wg, expert_wu, expert_wd, shared_wg, shared_wu, shared_wd)

    return out.reshape(orig_shape)
